# Initial kernel scaffold; baseline (speedup 1.0000x reference)
import jax
import jax.numpy as jnp
from jax.experimental import pallas as pl


def kernel(x, edge_index, all_edge_type, W, alpha_table, b):
    raise NotImplementedError("write your pallas kernel here")


# speedup vs baseline: 3.9378x; 3.9378x over previous
import functools

import jax
import jax.numpy as jnp
from jax import lax
from jax.experimental import pallas as pl
from jax.experimental.pallas import tpu as pltpu
from jax.experimental.pallas import tpu_sc as plsc


def _mm_body(x_ref, w_ref, o_ref):
    o_ref[...] = jnp.dot(x_ref[...], w_ref[...],
                         preferred_element_type=jnp.float32)


def _matmul(x, W):
    n, _ = x.shape
    d_out = W.shape[1]
    return pl.pallas_call(
        _mm_body,
        out_shape=jax.ShapeDtypeStruct((n, d_out), jnp.float32),
    )(x, W)


def _comb_body(p_ref, b_ref, o_ref):
    o_ref[...] = p_ref[0] + p_ref[1] + b_ref[...]


def _combine(partial, b):
    _, n, d_out = partial.shape
    return pl.pallas_call(
        _comb_body,
        out_shape=jax.ShapeDtypeStruct((n, d_out), jnp.float32),
    )(partial, b.reshape(1, d_out))


def _sc_scatter(feats, src, dst, et, ett, alpha_pad):
    n, d = feats.shape
    e = src.shape[0]
    info = plsc.get_sparse_core_info()
    nc, ns = info.num_cores, info.num_subcores
    nw = nc * ns
    ept = e // nw
    B = 80
    nchunk = ept // B
    nu = n // B
    upt = -(-nu // ns)
    ng = d // 16

    mesh = plsc.VectorSubcoreMesh(core_axis_name="c", subcore_axis_name="s")

    @functools.partial(
        pl.kernel,
        mesh=mesh,
        compiler_params=pltpu.CompilerParams(needs_layout_passes=False),
        out_type=jax.ShapeDtypeStruct((nc, n, d), jnp.float32),
        scratch_types=[
            pltpu.VMEM((B,), jnp.int32),
            pltpu.VMEM((B,), jnp.int32),
            pltpu.VMEM((B,), jnp.int32),
            pltpu.VMEM((B,), jnp.int32),
            pltpu.VMEM((32,), jnp.float32),
            pltpu.VMEM((B, d), jnp.float32),
            pltpu.VMEM_SHARED((n, d), jnp.float32),
            pltpu.SemaphoreType.DMA,
        ],
    )
    def k(feats_hbm, src_hbm, dst_hbm, et_hbm, ett_hbm, alpha_hbm, out_hbm,
          srci, dsti, eti, etti, alphav, rows, accum, sem):
        cid = lax.axis_index("c")
        sid = lax.axis_index("s")
        wid = sid * nc + cid

        zero16 = jnp.zeros((16,), jnp.float32)
        for r in range(B):
            for g in range(ng):
                rows[r, pl.ds(g * 16, 16)] = zero16
        for j in range(upt):
            u = sid + j * ns
            @pl.when(u < nu)
            def _():
                pltpu.sync_copy(rows, accum.at[pl.ds(pl.multiple_of(u * B, 16), B)])
        pltpu.sync_copy(alpha_hbm, alphav)
        plsc.subcore_barrier()

        at0 = alphav[pl.ds(0, 16)]
        at1 = alphav[pl.ds(16, 16)]

        def _splat(vec, lane):
            idx = jnp.full((16,), lane, jnp.int32)
            return vec.at[idx].get(mode="promise_in_bounds")

        def _lookup(v):
            lo = at0.at[jnp.minimum(v, 15)].get(mode="promise_in_bounds")
            hi = at1.at[jnp.maximum(v - 16, 0)].get(mode="promise_in_bounds")
            return jnp.where(v < 16, lo, hi)

        base = wid * ept

        def chunk(c, carry):
            off = base + c * B
            pltpu.sync_copy(src_hbm.at[pl.ds(off, B)], srci)
            pltpu.sync_copy(dst_hbm.at[pl.ds(off, B)], dsti)
            pltpu.sync_copy(et_hbm.at[pl.ds(off, B)], eti)
            pltpu.sync_copy(ett_hbm.at[pl.ds(off, B)], etti)
            pltpu.async_copy(feats_hbm.at[srci], rows, sem).wait()
            for j in range(B // 16):
                tv = eti[pl.ds(j * 16, 16)]
                ttv = etti[pl.ds(j * 16, 16)]
                av = _lookup(tv) + _lookup(ttv)
                for l in range(16):
                    r = j * 16 + l
                    s = _splat(av, l)
                    for g in range(ng):
                        rows[r, pl.ds(g * 16, 16)] = (
                            rows[r, pl.ds(g * 16, 16)] * s)
            pltpu.sync_copy(rows, accum.at[dsti], add=True)
            return carry

        lax.fori_loop(0, nchunk, chunk, 0)

        plsc.subcore_barrier()
        for j in range(upt):
            u = sid + j * ns
            @pl.when(u < nu)
            def _():
                r0 = pl.multiple_of(u * B, 16)
                pltpu.sync_copy(accum.at[pl.ds(r0, B)],
                                out_hbm.at[cid, pl.ds(r0, B)])

    return k(feats, src, dst, et, ett, alpha_pad)


def kernel(x, edge_index, all_edge_type, W, alpha_table, b):
    n = x.shape[0]
    e = all_edge_type.shape[0]
    t = (e - n) // 2
    src = edge_index[0]
    dst = edge_index[1]
    ett = jnp.concatenate([all_edge_type[t:2 * t],
                           all_edge_type[:t],
                           all_edge_type[2 * t:]])
    alpha_pad = jnp.pad(alpha_table[:, 0], (0, 32 - alpha_table.shape[0]))
    feats = _matmul(x, W)
    partial = _sc_scatter(feats, src, dst, all_edge_type, ett, alpha_pad)
    return _combine(partial, b)
